# initial kernel scaffold (unmeasured)
import jax
import jax.numpy as jnp
from jax import lax
from jax.experimental import pallas as pl
from jax.experimental.pallas import tpu as pltpu

N_DEV = 4


def _gelu(y):
    c = 0.7978845608028654
    return 0.5 * y * (1.0 + jnp.tanh(c * (y + 0.044715 * y * y * y)))


def kernel(x, w_mat):
    m, _ = x.shape
    _, n = w_mat.shape
    m_chunk = m // N_DEV

    def body(x_ref, w_ref, out_ref, comm_ref, send_sems, recv_sems):
        my = lax.axis_index("i")
        left = lax.rem(my + (N_DEV - 1), N_DEV)
        right = lax.rem(my + 1, N_DEV)

        barrier_sem = pltpu.get_barrier_semaphore()
        for nbr in (left, right):
            pl.semaphore_signal(
                barrier_sem, inc=1,
                device_id=(nbr,), device_id_type=pl.DeviceIdType.MESH,
            )
        pl.semaphore_wait(barrier_sem, 2)

        def local_chunk(c):
            return jnp.dot(
                x_ref[pl.ds(c * m_chunk, m_chunk), :],
                w_ref[...],
                preferred_element_type=jnp.float32,
            )

        comm_ref[0, :, :] = local_chunk(lax.rem(my + (N_DEV - 1), N_DEV))

        for h in range(N_DEV - 1):
            rdma = pltpu.make_async_remote_copy(
                src_ref=comm_ref.at[h],
                dst_ref=comm_ref.at[h + 1],
                send_sem=send_sems.at[h],
                recv_sem=recv_sems.at[h],
                device_id=(right,),
                device_id_type=pl.DeviceIdType.MESH,
            )
            rdma.start()
            rdma.wait()
            c = lax.rem(my + (2 * N_DEV - 2 - h), N_DEV)
            acc = comm_ref[h + 1, :, :] + local_chunk(c)
            if h < N_DEV - 2:
                comm_ref[h + 1, :, :] = acc
            else:
                out_ref[...] = _gelu(acc)

    return pl.pallas_call(
        body,
        out_shape=jax.ShapeDtypeStruct((m_chunk, n), jnp.float32),
        in_specs=[
            pl.BlockSpec(memory_space=pltpu.VMEM),
            pl.BlockSpec(memory_space=pltpu.VMEM),
        ],
        out_specs=pl.BlockSpec(memory_space=pltpu.VMEM),
        scratch_shapes=[
            pltpu.VMEM((N_DEV, m_chunk, n), jnp.float32),
            pltpu.SemaphoreType.DMA((N_DEV - 1,)),
            pltpu.SemaphoreType.DMA((N_DEV - 1,)),
        ],
        compiler_params=pltpu.CompilerParams(collective_id=0),
    )(x, w_mat)


# baseline (device time: 317514 ns/iter reference)
import jax
import jax.numpy as jnp
from jax import lax
from jax.experimental import pallas as pl
from jax.experimental.pallas import tpu as pltpu

N_DEV = 4


def _gelu(y):
    c = 0.7978845608028654
    return 0.5 * y * (1.0 + jnp.tanh(c * (y + 0.044715 * y * y * y)))


def kernel(x, w_mat):
    m, _ = x.shape
    _, n = w_mat.shape
    m_chunk = m // N_DEV

    def body(x_ref, w_ref, out_ref, comm_ref, send_sems, recv_sems):
        my = lax.axis_index("i")
        left = lax.rem(my + (N_DEV - 1), N_DEV)
        right = lax.rem(my + 1, N_DEV)

        barrier_sem = pltpu.get_barrier_semaphore()
        for nbr in (left, right):
            pl.semaphore_signal(
                barrier_sem, inc=1,
                device_id=(nbr,), device_id_type=pl.DeviceIdType.MESH,
            )
        pl.semaphore_wait(barrier_sem, 2)

        def local_chunk(c):
            return jnp.dot(
                x_ref[pl.ds(c * m_chunk, m_chunk), :],
                w_ref[...],
                preferred_element_type=jnp.float32,
            )

        comm_ref[0, :, :] = local_chunk(lax.rem(my + (N_DEV - 1), N_DEV))

        for h in range(N_DEV - 1):
            last = h == N_DEV - 2
            rdma = pltpu.make_async_remote_copy(
                src_ref=comm_ref.at[h],
                dst_ref=out_ref if last else comm_ref.at[h + 1],
                send_sem=send_sems.at[h],
                recv_sem=recv_sems.at[h],
                device_id=(right,),
                device_id_type=pl.DeviceIdType.MESH,
            )
            rdma.start()
            rdma.wait()
            c = lax.rem(my + (2 * N_DEV - 2 - h), N_DEV)
            if not last:
                comm_ref[h + 1, :, :] = comm_ref[h + 1, :, :] + local_chunk(c)
            else:
                out_ref[...] = _gelu(out_ref[...] + local_chunk(c))

    return pl.pallas_call(
        body,
        out_shape=jax.ShapeDtypeStruct((m_chunk, n), jnp.float32),
        in_specs=[
            pl.BlockSpec(memory_space=pltpu.VMEM),
            pl.BlockSpec(memory_space=pltpu.VMEM),
        ],
        out_specs=pl.BlockSpec(memory_space=pltpu.VMEM),
        scratch_shapes=[
            pltpu.VMEM((N_DEV - 1, m_chunk, n), jnp.float32),
            pltpu.SemaphoreType.DMA((N_DEV - 1,)),
            pltpu.SemaphoreType.DMA((N_DEV - 1,)),
        ],
        compiler_params=pltpu.CompilerParams(
            collective_id=0,
            vmem_limit_bytes=100 * 1024 * 1024,
        ),
    )(x, w_mat)


# device time: 182587 ns/iter; 1.7390x vs baseline; 1.7390x over previous
import jax
import jax.numpy as jnp
from jax import lax
from jax.experimental import pallas as pl
from jax.experimental.pallas import tpu as pltpu

N_DEV = 4


def _gelu(y):
    c = 0.7978845608028654
    return 0.5 * y * (1.0 + jnp.tanh(c * (y + 0.044715 * y * y * y)))


def kernel(x, w_mat):
    m, _ = x.shape
    _, n = w_mat.shape
    m_chunk = m // N_DEV
    n_half = n // 2

    def body(x_ref, w_ref, out_ref, cw_ref, ccw_ref,
             cw_send, cw_recv, ccw_send, ccw_recv):
        my = lax.axis_index("i")
        left = lax.rem(my + (N_DEV - 1), N_DEV)
        right = lax.rem(my + 1, N_DEV)

        barrier_sem = pltpu.get_barrier_semaphore()
        for nbr in (left, right):
            pl.semaphore_signal(
                barrier_sem, inc=1,
                device_id=(nbr,), device_id_type=pl.DeviceIdType.MESH,
            )
        pl.semaphore_wait(barrier_sem, 2)

        def cw_gemm(c):
            return jnp.dot(
                x_ref[pl.ds(c * m_chunk, m_chunk), :],
                w_ref[:, :n_half],
                preferred_element_type=jnp.float32,
            )

        def ccw_gemm(c):
            return jnp.dot(
                x_ref[pl.ds(c * m_chunk, m_chunk), :],
                w_ref[:, n_half:],
                preferred_element_type=jnp.float32,
            )

        cw_ref[0, :, :] = cw_gemm(lax.rem(my + (N_DEV - 1), N_DEV))
        ccw_ref[0, :, :] = ccw_gemm(lax.rem(my + 1, N_DEV))

        for h in range(N_DEV - 1):
            last = h == N_DEV - 2
            rdma_cw = pltpu.make_async_remote_copy(
                src_ref=cw_ref.at[h],
                dst_ref=out_ref.at[:, :n_half] if last else cw_ref.at[h + 1],
                send_sem=cw_send.at[h],
                recv_sem=cw_recv.at[h],
                device_id=(right,),
                device_id_type=pl.DeviceIdType.MESH,
            )
            rdma_ccw = pltpu.make_async_remote_copy(
                src_ref=ccw_ref.at[h],
                dst_ref=out_ref.at[:, n_half:] if last else ccw_ref.at[h + 1],
                send_sem=ccw_send.at[h],
                recv_sem=ccw_recv.at[h],
                device_id=(left,),
                device_id_type=pl.DeviceIdType.MESH,
            )
            rdma_cw.start()
            rdma_ccw.start()
            rdma_cw.wait()
            rdma_ccw.wait()
            c_cw = lax.rem(my + (2 * N_DEV - 2 - h), N_DEV)
            c_ccw = lax.rem(my + (2 + h), N_DEV)
            if not last:
                cw_ref[h + 1, :, :] = cw_ref[h + 1, :, :] + cw_gemm(c_cw)
                ccw_ref[h + 1, :, :] = ccw_ref[h + 1, :, :] + ccw_gemm(c_ccw)
            else:
                out_ref[:, :n_half] = _gelu(out_ref[:, :n_half] + cw_gemm(c_cw))
                out_ref[:, n_half:] = _gelu(out_ref[:, n_half:] + ccw_gemm(c_ccw))

    return pl.pallas_call(
        body,
        out_shape=jax.ShapeDtypeStruct((m_chunk, n), jnp.float32),
        in_specs=[
            pl.BlockSpec(memory_space=pltpu.VMEM),
            pl.BlockSpec(memory_space=pltpu.VMEM),
        ],
        out_specs=pl.BlockSpec(memory_space=pltpu.VMEM),
        scratch_shapes=[
            pltpu.VMEM((N_DEV - 1, m_chunk, n_half), jnp.float32),
            pltpu.VMEM((N_DEV - 1, m_chunk, n_half), jnp.float32),
            pltpu.SemaphoreType.DMA((N_DEV - 1,)),
            pltpu.SemaphoreType.DMA((N_DEV - 1,)),
            pltpu.SemaphoreType.DMA((N_DEV - 1,)),
            pltpu.SemaphoreType.DMA((N_DEV - 1,)),
        ],
        compiler_params=pltpu.CompilerParams(
            collective_id=0,
            vmem_limit_bytes=100 * 1024 * 1024,
        ),
    )(x, w_mat)


# device time: 175562 ns/iter; 1.8086x vs baseline; 1.0400x over previous
import jax
import jax.numpy as jnp
from jax import lax
from jax.experimental import pallas as pl
from jax.experimental.pallas import tpu as pltpu

N_DEV = 4


def _gelu(y):
    c = 0.7978845608028654
    return 0.5 * y * (1.0 + jnp.tanh(c * (y + 0.044715 * y * y * y)))


def kernel(x, w_mat):
    m, _ = x.shape
    _, n = w_mat.shape
    m_chunk = m // N_DEV
    n_half = n // 2

    def body(x_ref, w_ref, out_ref, cw_ref, ccw_ref,
             cw_send, cw_recv, ccw_send, ccw_recv):
        my = lax.axis_index("i")
        left = lax.rem(my + (N_DEV - 1), N_DEV)
        right = lax.rem(my + 1, N_DEV)

        barrier_sem = pltpu.get_barrier_semaphore()
        for nbr in (left, right):
            pl.semaphore_signal(
                barrier_sem, inc=1,
                device_id=(nbr,), device_id_type=pl.DeviceIdType.MESH,
            )
        pl.semaphore_wait(barrier_sem, 2)

        def cw_gemm(c):
            return jnp.dot(
                x_ref[pl.ds(c * m_chunk, m_chunk), :],
                w_ref[:, :n_half],
                preferred_element_type=jnp.float32,
            )

        def ccw_gemm(c):
            return jnp.dot(
                x_ref[pl.ds(c * m_chunk, m_chunk), :],
                w_ref[:, n_half:],
                preferred_element_type=jnp.float32,
            )

        def make_cw(h):
            last = h == N_DEV - 2
            return pltpu.make_async_remote_copy(
                src_ref=cw_ref.at[h],
                dst_ref=out_ref.at[:, :n_half] if last else cw_ref.at[h + 1],
                send_sem=cw_send.at[h],
                recv_sem=cw_recv.at[h],
                device_id=(right,),
                device_id_type=pl.DeviceIdType.MESH,
            )

        def make_ccw(h):
            last = h == N_DEV - 2
            return pltpu.make_async_remote_copy(
                src_ref=ccw_ref.at[h],
                dst_ref=out_ref.at[:, n_half:] if last else ccw_ref.at[h + 1],
                send_sem=ccw_send.at[h],
                recv_sem=ccw_recv.at[h],
                device_id=(left,),
                device_id_type=pl.DeviceIdType.MESH,
            )

        sends = []

        cw_ref[0, :, :] = cw_gemm(lax.rem(my + (N_DEV - 1), N_DEV))
        r = make_cw(0)
        r.start()
        sends.append(r)
        ccw_ref[0, :, :] = ccw_gemm(lax.rem(my + 1, N_DEV))
        r = make_ccw(0)
        r.start()
        sends.append(r)

        for h in range(N_DEV - 1):
            last = h == N_DEV - 2
            c_cw = lax.rem(my + (2 * N_DEV - 2 - h), N_DEV)
            c_ccw = lax.rem(my + (2 + h), N_DEV)

            make_cw(h).wait_recv()
            if not last:
                cw_ref[h + 1, :, :] = cw_ref[h + 1, :, :] + cw_gemm(c_cw)
                r = make_cw(h + 1)
                r.start()
                sends.append(r)
            else:
                out_ref[:, :n_half] = _gelu(out_ref[:, :n_half] + cw_gemm(c_cw))

            make_ccw(h).wait_recv()
            if not last:
                ccw_ref[h + 1, :, :] = ccw_ref[h + 1, :, :] + ccw_gemm(c_ccw)
                r = make_ccw(h + 1)
                r.start()
                sends.append(r)
            else:
                out_ref[:, n_half:] = _gelu(out_ref[:, n_half:] + ccw_gemm(c_ccw))

        for r in sends:
            r.wait_send()

    return pl.pallas_call(
        body,
        out_shape=jax.ShapeDtypeStruct((m_chunk, n), jnp.float32),
        in_specs=[
            pl.BlockSpec(memory_space=pltpu.VMEM),
            pl.BlockSpec(memory_space=pltpu.VMEM),
        ],
        out_specs=pl.BlockSpec(memory_space=pltpu.VMEM),
        scratch_shapes=[
            pltpu.VMEM((N_DEV - 1, m_chunk, n_half), jnp.float32),
            pltpu.VMEM((N_DEV - 1, m_chunk, n_half), jnp.float32),
            pltpu.SemaphoreType.DMA((N_DEV - 1,)),
            pltpu.SemaphoreType.DMA((N_DEV - 1,)),
            pltpu.SemaphoreType.DMA((N_DEV - 1,)),
            pltpu.SemaphoreType.DMA((N_DEV - 1,)),
        ],
        compiler_params=pltpu.CompilerParams(
            collective_id=0,
            vmem_limit_bytes=100 * 1024 * 1024,
        ),
    )(x, w_mat)


# device time: 163017 ns/iter; 1.9477x vs baseline; 1.0770x over previous
import jax
import jax.numpy as jnp
from jax import lax
from jax.experimental import pallas as pl
from jax.experimental.pallas import tpu as pltpu

N_DEV = 4
SUB = 2


def _gelu(y):
    c = 0.7978845608028654
    return 0.5 * y * (1.0 + jnp.tanh(c * (y + 0.044715 * y * y * y)))


def kernel(x, w_mat):
    m, _ = x.shape
    _, n = w_mat.shape
    m_chunk = m // N_DEV
    n_half = n // 2
    m_sub = m_chunk // SUB

    def body(x_ref, w_ref, out_ref, cw_ref, ccw_ref,
             cw_send, cw_recv, ccw_send, ccw_recv):
        my = lax.axis_index("i")
        left = lax.rem(my + (N_DEV - 1), N_DEV)
        right = lax.rem(my + 1, N_DEV)

        barrier_sem = pltpu.get_barrier_semaphore()
        for nbr in (left, right):
            pl.semaphore_signal(
                barrier_sem, inc=1,
                device_id=(nbr,), device_id_type=pl.DeviceIdType.MESH,
            )
        pl.semaphore_wait(barrier_sem, 2)

        def sub_gemm(c, s, lo):
            cols = slice(None, n_half) if lo else slice(n_half, None)
            return jnp.dot(
                x_ref[pl.ds(c * m_chunk + s * m_sub, m_sub), :],
                w_ref[:, cols],
                preferred_element_type=jnp.float32,
            )

        def make_cw(h, s):
            last = h == N_DEV - 2
            rows = pl.ds(s * m_sub, m_sub)
            return pltpu.make_async_remote_copy(
                src_ref=cw_ref.at[h, rows, :],
                dst_ref=(out_ref.at[rows, :n_half] if last
                         else cw_ref.at[h + 1, rows, :]),
                send_sem=cw_send.at[h * SUB + s],
                recv_sem=cw_recv.at[h * SUB + s],
                device_id=(right,),
                device_id_type=pl.DeviceIdType.MESH,
            )

        def make_ccw(h, s):
            last = h == N_DEV - 2
            rows = pl.ds(s * m_sub, m_sub)
            return pltpu.make_async_remote_copy(
                src_ref=ccw_ref.at[h, rows, :],
                dst_ref=(out_ref.at[rows, n_half:] if last
                         else ccw_ref.at[h + 1, rows, :]),
                send_sem=ccw_send.at[h * SUB + s],
                recv_sem=ccw_recv.at[h * SUB + s],
                device_id=(left,),
                device_id_type=pl.DeviceIdType.MESH,
            )

        sends = []

        def start(r):
            r.start()
            sends.append(r)

        c0_cw = lax.rem(my + (N_DEV - 1), N_DEV)
        c0_ccw = lax.rem(my + 1, N_DEV)
        for s in range(SUB):
            rows = pl.ds(s * m_sub, m_sub)
            cw_ref[0, rows, :] = sub_gemm(c0_cw, s, True)
            start(make_cw(0, s))
            ccw_ref[0, rows, :] = sub_gemm(c0_ccw, s, False)
            start(make_ccw(0, s))

        for h in range(N_DEV - 1):
            last = h == N_DEV - 2
            c_cw = lax.rem(my + (2 * N_DEV - 2 - h), N_DEV)
            c_ccw = lax.rem(my + (2 + h), N_DEV)
            for s in range(SUB):
                rows = pl.ds(s * m_sub, m_sub)

                make_cw(h, s).wait_recv()
                if not last:
                    cw_ref[h + 1, rows, :] = (
                        cw_ref[h + 1, rows, :] + sub_gemm(c_cw, s, True)
                    )
                    start(make_cw(h + 1, s))
                else:
                    out_ref[rows, :n_half] = _gelu(
                        out_ref[rows, :n_half] + sub_gemm(c_cw, s, True)
                    )

                make_ccw(h, s).wait_recv()
                if not last:
                    ccw_ref[h + 1, rows, :] = (
                        ccw_ref[h + 1, rows, :] + sub_gemm(c_ccw, s, False)
                    )
                    start(make_ccw(h + 1, s))
                else:
                    out_ref[rows, n_half:] = _gelu(
                        out_ref[rows, n_half:] + sub_gemm(c_ccw, s, False)
                    )

        for r in sends:
            r.wait_send()

    return pl.pallas_call(
        body,
        out_shape=jax.ShapeDtypeStruct((m_chunk, n), jnp.float32),
        in_specs=[
            pl.BlockSpec(memory_space=pltpu.VMEM),
            pl.BlockSpec(memory_space=pltpu.VMEM),
        ],
        out_specs=pl.BlockSpec(memory_space=pltpu.VMEM),
        scratch_shapes=[
            pltpu.VMEM((N_DEV - 1, m_chunk, n_half), jnp.float32),
            pltpu.VMEM((N_DEV - 1, m_chunk, n_half), jnp.float32),
            pltpu.SemaphoreType.DMA(((N_DEV - 1) * SUB,)),
            pltpu.SemaphoreType.DMA(((N_DEV - 1) * SUB,)),
            pltpu.SemaphoreType.DMA(((N_DEV - 1) * SUB,)),
            pltpu.SemaphoreType.DMA(((N_DEV - 1) * SUB,)),
        ],
        compiler_params=pltpu.CompilerParams(
            collective_id=0,
            vmem_limit_bytes=100 * 1024 * 1024,
        ),
    )(x, w_mat)


# device time: 161127 ns/iter; 1.9706x vs baseline; 1.0117x over previous
import jax
import jax.numpy as jnp
from jax import lax
from jax.experimental import pallas as pl
from jax.experimental.pallas import tpu as pltpu

N_DEV = 4
SUB = 4


def _gelu(y):
    c = 0.7978845608028654
    return 0.5 * y * (1.0 + jnp.tanh(c * (y + 0.044715 * y * y * y)))


def kernel(x, w_mat):
    m, _ = x.shape
    _, n = w_mat.shape
    m_chunk = m // N_DEV
    n_half = n // 2
    m_sub = m_chunk // SUB

    def body(x_ref, w_ref, out_ref, cw_ref, ccw_ref,
             cw_send, cw_recv, ccw_send, ccw_recv):
        my = lax.axis_index("i")
        left = lax.rem(my + (N_DEV - 1), N_DEV)
        right = lax.rem(my + 1, N_DEV)

        barrier_sem = pltpu.get_barrier_semaphore()
        for nbr in (left, right):
            pl.semaphore_signal(
                barrier_sem, inc=1,
                device_id=(nbr,), device_id_type=pl.DeviceIdType.MESH,
            )
        pl.semaphore_wait(barrier_sem, 2)

        def sub_gemm(c, s, lo):
            cols = slice(None, n_half) if lo else slice(n_half, None)
            return jnp.dot(
                x_ref[pl.ds(c * m_chunk + s * m_sub, m_sub), :],
                w_ref[:, cols],
                preferred_element_type=jnp.float32,
            )

        def make_cw(h, s):
            last = h == N_DEV - 2
            rows = pl.ds(s * m_sub, m_sub)
            return pltpu.make_async_remote_copy(
                src_ref=cw_ref.at[h, rows, :],
                dst_ref=(out_ref.at[rows, :n_half] if last
                         else cw_ref.at[h + 1, rows, :]),
                send_sem=cw_send.at[h * SUB + s],
                recv_sem=cw_recv.at[h * SUB + s],
                device_id=(right,),
                device_id_type=pl.DeviceIdType.MESH,
            )

        def make_ccw(h, s):
            last = h == N_DEV - 2
            rows = pl.ds(s * m_sub, m_sub)
            return pltpu.make_async_remote_copy(
                src_ref=ccw_ref.at[h, rows, :],
                dst_ref=(out_ref.at[rows, n_half:] if last
                         else ccw_ref.at[h + 1, rows, :]),
                send_sem=ccw_send.at[h * SUB + s],
                recv_sem=ccw_recv.at[h * SUB + s],
                device_id=(left,),
                device_id_type=pl.DeviceIdType.MESH,
            )

        sends = []

        def start(r):
            r.start()
            sends.append(r)

        c0_cw = lax.rem(my + (N_DEV - 1), N_DEV)
        c0_ccw = lax.rem(my + 1, N_DEV)
        for s in range(SUB):
            rows = pl.ds(s * m_sub, m_sub)
            cw_ref[0, rows, :] = sub_gemm(c0_cw, s, True)
            start(make_cw(0, s))
            ccw_ref[0, rows, :] = sub_gemm(c0_ccw, s, False)
            start(make_ccw(0, s))

        for h in range(N_DEV - 1):
            last = h == N_DEV - 2
            c_cw = lax.rem(my + (2 * N_DEV - 2 - h), N_DEV)
            c_ccw = lax.rem(my + (2 + h), N_DEV)
            for s in range(SUB):
                rows = pl.ds(s * m_sub, m_sub)

                make_cw(h, s).wait_recv()
                if not last:
                    cw_ref[h + 1, rows, :] = (
                        cw_ref[h + 1, rows, :] + sub_gemm(c_cw, s, True)
                    )
                    start(make_cw(h + 1, s))
                else:
                    out_ref[rows, :n_half] = _gelu(
                        out_ref[rows, :n_half] + sub_gemm(c_cw, s, True)
                    )

                make_ccw(h, s).wait_recv()
                if not last:
                    ccw_ref[h + 1, rows, :] = (
                        ccw_ref[h + 1, rows, :] + sub_gemm(c_ccw, s, False)
                    )
                    start(make_ccw(h + 1, s))
                else:
                    out_ref[rows, n_half:] = _gelu(
                        out_ref[rows, n_half:] + sub_gemm(c_ccw, s, False)
                    )

        for r in sends:
            r.wait_send()

    return pl.pallas_call(
        body,
        out_shape=jax.ShapeDtypeStruct((m_chunk, n), jnp.float32),
        in_specs=[
            pl.BlockSpec(memory_space=pltpu.VMEM),
            pl.BlockSpec(memory_space=pltpu.VMEM),
        ],
        out_specs=pl.BlockSpec(memory_space=pltpu.VMEM),
        scratch_shapes=[
            pltpu.VMEM((N_DEV - 1, m_chunk, n_half), jnp.float32),
            pltpu.VMEM((N_DEV - 1, m_chunk, n_half), jnp.float32),
            pltpu.SemaphoreType.DMA(((N_DEV - 1) * SUB,)),
            pltpu.SemaphoreType.DMA(((N_DEV - 1) * SUB,)),
            pltpu.SemaphoreType.DMA(((N_DEV - 1) * SUB,)),
            pltpu.SemaphoreType.DMA(((N_DEV - 1) * SUB,)),
        ],
        compiler_params=pltpu.CompilerParams(
            collective_id=0,
            vmem_limit_bytes=100 * 1024 * 1024,
        ),
    )(x, w_mat)


# device time: 93746 ns/iter; 3.3870x vs baseline; 1.7188x over previous
import jax
import jax.numpy as jnp
from jax import lax
from jax.experimental import pallas as pl
from jax.experimental.pallas import tpu as pltpu

N_DEV = 4
SUB = 4


def _gelu(y):
    c = 0.7978845608028654
    return 0.5 * y * (1.0 + jnp.tanh(c * (y + 0.044715 * y * y * y)))


def kernel(x, w_mat):
    m, _ = x.shape
    _, n = w_mat.shape
    m_chunk = m // N_DEV
    n_half = n // 2
    m_sub = m_chunk // SUB

    def body(x_ref, w_ref, out_ref, cw_ref, ccw_ref,
             cw_send, cw_recv, ccw_send, ccw_recv):
        my = lax.axis_index("i")
        left = lax.rem(my + (N_DEV - 1), N_DEV)
        right = lax.rem(my + 1, N_DEV)

        barrier_sem = pltpu.get_barrier_semaphore()
        for nbr in (left, right):
            pl.semaphore_signal(
                barrier_sem, inc=1,
                device_id=(nbr,), device_id_type=pl.DeviceIdType.MESH,
            )
        pl.semaphore_wait(barrier_sem, 2)

        def sub_gemm(c, s, lo):
            cols = slice(None, n_half) if lo else slice(n_half, None)
            return jnp.dot(
                x_ref[pl.ds(c * m_chunk + s * m_sub, m_sub), :],
                w_ref[:, cols],
                preferred_element_type=jnp.float32,
            )

        def make_rdma(dir_ref, send_sems, recv_sems, target, h, s):
            rows = pl.ds(s * m_sub, m_sub)
            return pltpu.make_async_remote_copy(
                src_ref=dir_ref.at[h, rows, :],
                dst_ref=dir_ref.at[h + 1, rows, :],
                send_sem=send_sems.at[h * SUB + s],
                recv_sem=recv_sems.at[h * SUB + s],
                device_id=(target,),
                device_id_type=pl.DeviceIdType.MESH,
            )

        def make_cw(h, s):
            return make_rdma(cw_ref, cw_send, cw_recv, right, h, s)

        def make_ccw(h, s):
            return make_rdma(ccw_ref, ccw_send, ccw_recv, left, h, s)

        sends = []

        def start(r):
            r.start()
            sends.append(r)

        c0_cw = lax.rem(my + (N_DEV - 1), N_DEV)
        c0_ccw = lax.rem(my + 1, N_DEV)
        for s in range(SUB):
            rows = pl.ds(s * m_sub, m_sub)
            cw_ref[0, rows, :] = sub_gemm(c0_cw, s, True).astype(jnp.bfloat16)
            start(make_cw(0, s))
            ccw_ref[0, rows, :] = sub_gemm(c0_ccw, s, False).astype(jnp.bfloat16)
            start(make_ccw(0, s))

        for h in range(N_DEV - 1):
            last = h == N_DEV - 2
            c_cw = lax.rem(my + (2 * N_DEV - 2 - h), N_DEV)
            c_ccw = lax.rem(my + (2 + h), N_DEV)
            for s in range(SUB):
                rows = pl.ds(s * m_sub, m_sub)

                make_cw(h, s).wait_recv()
                acc = (cw_ref[h + 1, rows, :].astype(jnp.float32)
                       + sub_gemm(c_cw, s, True))
                if not last:
                    cw_ref[h + 1, rows, :] = acc.astype(jnp.bfloat16)
                    start(make_cw(h + 1, s))
                else:
                    out_ref[rows, :n_half] = _gelu(acc)

                make_ccw(h, s).wait_recv()
                acc = (ccw_ref[h + 1, rows, :].astype(jnp.float32)
                       + sub_gemm(c_ccw, s, False))
                if not last:
                    ccw_ref[h + 1, rows, :] = acc.astype(jnp.bfloat16)
                    start(make_ccw(h + 1, s))
                else:
                    out_ref[rows, n_half:] = _gelu(acc)

        for r in sends:
            r.wait_send()

    return pl.pallas_call(
        body,
        out_shape=jax.ShapeDtypeStruct((m_chunk, n), jnp.float32),
        in_specs=[
            pl.BlockSpec(memory_space=pltpu.VMEM),
            pl.BlockSpec(memory_space=pltpu.VMEM),
        ],
        out_specs=pl.BlockSpec(memory_space=pltpu.VMEM),
        scratch_shapes=[
            pltpu.VMEM((N_DEV, m_chunk, n_half), jnp.bfloat16),
            pltpu.VMEM((N_DEV, m_chunk, n_half), jnp.bfloat16),
            pltpu.SemaphoreType.DMA(((N_DEV - 1) * SUB,)),
            pltpu.SemaphoreType.DMA(((N_DEV - 1) * SUB,)),
            pltpu.SemaphoreType.DMA(((N_DEV - 1) * SUB,)),
            pltpu.SemaphoreType.DMA(((N_DEV - 1) * SUB,)),
        ],
        compiler_params=pltpu.CompilerParams(
            collective_id=0,
            vmem_limit_bytes=100 * 1024 * 1024,
        ),
    )(x, w_mat)
